# baseline (speedup 1.0000x reference)
import jax
import jax.numpy as jnp
from jax import lax
from jax.experimental import pallas as pl

_F = 16
_CUTOFF = 5.0
_BLOCK_E = 16000

_SIN_C = (
    3.141592653589793,
    -5.167712780049970,
    2.550164039877345,
    -0.5992645293207921,
    0.08214588661112823,
    -0.007370430945714350,
)


def _sin_pi(z):
    k = (z + 0.5).astype(jnp.int32)
    w = z - k.astype(jnp.float32)
    t = w * w
    p = _SIN_C[5]
    for c in (_SIN_C[4], _SIN_C[3], _SIN_C[2], _SIN_C[1], _SIN_C[0]):
        p = p * t + c
    s = w * p
    sbits = lax.shift_left(lax.bitwise_and(k, 1), 31)
    return lax.bitcast_convert_type(
        lax.bitwise_xor(lax.bitcast_convert_type(s, jnp.int32), sbits),
        jnp.float32,
    )


def _body(r_ref, x_ref, y_ref, w_ref, o_ref):
    rt = r_ref[0]
    d = jnp.clip(rt * _CUTOFF, 1e-6, _CUTOFF)
    u = d / _CUTOFF
    env = 1.0 - 6.0 * u ** 5 + 15.0 * u ** 4 - 10.0 * u ** 3
    g = jnp.sqrt(2.0 / _CUTOFF) * env / d
    n = lax.broadcasted_iota(jnp.int32, (_F, 1), 0).astype(jnp.float32) + 1.0
    z = n * u
    s = _sin_pi(z) * g
    w = w_ref[0]
    out_ten = lax.dot_general(
        s, w, (((0,), (0,)), ((), ())),
        preferred_element_type=jnp.float32,
    )
    o_ref[:, :] = x_ref[:, :] * y_ref[:, :] * out_ten


def kernel(x, y, r, W):
    E, C = x.shape
    nb = E // _BLOCK_E
    r3 = r.reshape(nb, 1, _BLOCK_E)
    return pl.pallas_call(
        _body,
        grid=(nb,),
        in_specs=[
            pl.BlockSpec((1, 1, _BLOCK_E), lambda i: (i, 0, 0)),
            pl.BlockSpec((_BLOCK_E, C), lambda i: (i, 0)),
            pl.BlockSpec((_BLOCK_E, C), lambda i: (i, 0)),
            pl.BlockSpec((1, _F, C), lambda i: (0, 0, 0)),
        ],
        out_specs=pl.BlockSpec((_BLOCK_E, C), lambda i: (i, 0)),
        out_shape=jax.ShapeDtypeStruct((E, C), x.dtype),
    )(r3, x, y, W)


# speedup vs baseline: 1.0039x; 1.0039x over previous
import jax
import jax.numpy as jnp
from jax import lax
from jax.experimental import pallas as pl

_F = 16
_CUTOFF = 5.0
_BLOCK_E = 12800

_SIN_C = (
    3.141592653589793,
    -5.167712780049970,
    2.550164039877345,
    -0.5992645293207921,
    0.08214588661112823,
    -0.007370430945714350,
)


def _sin_pi(z):
    k = (z + 0.5).astype(jnp.int32)
    w = z - k.astype(jnp.float32)
    t = w * w
    p = _SIN_C[5]
    for c in (_SIN_C[4], _SIN_C[3], _SIN_C[2], _SIN_C[1], _SIN_C[0]):
        p = p * t + c
    s = w * p
    sbits = lax.shift_left(lax.bitwise_and(k, 1), 31)
    return lax.bitcast_convert_type(
        lax.bitwise_xor(lax.bitcast_convert_type(s, jnp.int32), sbits),
        jnp.float32,
    )


def _body(r_ref, x_ref, y_ref, w_ref, o_ref):
    rt = r_ref[0]
    d = jnp.clip(rt * _CUTOFF, 1e-6, _CUTOFF)
    u = d / _CUTOFF
    env = 1.0 - 6.0 * u ** 5 + 15.0 * u ** 4 - 10.0 * u ** 3
    g = jnp.sqrt(2.0 / _CUTOFF) * env / d
    n = lax.broadcasted_iota(jnp.int32, (_F, 1), 0).astype(jnp.float32) + 1.0
    z = n * u
    s = _sin_pi(z) * g
    w = w_ref[0]
    out_ten = lax.dot_general(
        s, w, (((0,), (0,)), ((), ())),
        preferred_element_type=jnp.float32,
    )
    o_ref[:, :] = x_ref[:, :] * y_ref[:, :] * out_ten


def kernel(x, y, r, W):
    E, C = x.shape
    nb = E // _BLOCK_E
    r3 = r.reshape(nb, 1, _BLOCK_E)
    return pl.pallas_call(
        _body,
        grid=(nb,),
        in_specs=[
            pl.BlockSpec((1, 1, _BLOCK_E), lambda i: (i, 0, 0)),
            pl.BlockSpec((_BLOCK_E, C), lambda i: (i, 0)),
            pl.BlockSpec((_BLOCK_E, C), lambda i: (i, 0)),
            pl.BlockSpec((1, _F, C), lambda i: (0, 0, 0)),
        ],
        out_specs=pl.BlockSpec((_BLOCK_E, C), lambda i: (i, 0)),
        out_shape=jax.ShapeDtypeStruct((E, C), x.dtype),
    )(r3, x, y, W)
